# baseline (speedup 1.0000x reference)
import functools

import jax
import jax.numpy as jnp
from jax import lax
from jax.experimental import pallas as pl
from jax.experimental.pallas import tpu as pltpu
from jax.experimental.pallas import tpu_sc as plsc

VOCAB = 60000
EMBED_DIM = 128
SEQ = 50
NSENT = 16384
SEQ_PAD = 64
NBUF = 4
SLABS = 4

_info = plsc.get_sparse_core_info()
_NC, _NS = _info.num_cores, _info.num_subcores
_NW = _NC * _NS

_CH = 4

_mesh = plsc.VectorSubcoreMesh(core_axis_name="c", subcore_axis_name="s")


def _make_gather_kernel(nsent):
    per_w = nsent // _NW
    steps = per_w // _CH

    @functools.partial(
        pl.kernel,
        mesh=_mesh,
        out_type=jax.ShapeDtypeStruct((nsent, SEQ, EMBED_DIM), jnp.float32),
        scratch_types=[
            pltpu.VMEM((NBUF, _CH, SEQ_PAD), jnp.int32),
            pltpu.VMEM((NBUF, _CH, SEQ, EMBED_DIM), jnp.float32),
            [pltpu.SemaphoreType.DMA] * NBUF,
            [pltpu.SemaphoreType.DMA] * NBUF,
            [pltpu.SemaphoreType.DMA] * NBUF,
        ],
    )
    def gather_kernel(idx_hbm, table_hbm, out_hbm, idx_v, rows_v, sg, so, si):
        wid = lax.axis_index("s") * _NC + lax.axis_index("c")
        base_sent = wid * per_w

        def fetch_idx(c, b):
            pltpu.async_copy(
                idx_hbm.at[pl.ds(base_sent + c * _CH, _CH)], idx_v.at[b], si[b]
            )

        def fire_gathers(c, b):
            pltpu.make_async_copy(
                idx_hbm.at[pl.ds(base_sent, _CH)], idx_v.at[b], si[b]
            ).wait()
            for j in range(_CH):
                pltpu.async_copy(
                    table_hbm.at[idx_v.at[b, j, pl.ds(0, SEQ)]],
                    rows_v.at[b, j],
                    sg[b],
                )

        def retire_chunk(c, b, last):
            for _ in range(_CH):
                pltpu.make_async_copy(
                    table_hbm.at[idx_v.at[b, 0, pl.ds(0, SEQ)]],
                    rows_v.at[b, 0],
                    sg[b],
                ).wait()
            if not last:
                @pl.when(c + NBUF < steps)
                def _():
                    fetch_idx(c + NBUF, b)

            pltpu.async_copy(
                rows_v.at[b], out_hbm.at[pl.ds(base_sent + c * _CH, _CH)], so[b]
            )

        def wait_write(b):
            pltpu.make_async_copy(
                rows_v.at[b], out_hbm.at[pl.ds(base_sent, _CH)], so[b]
            ).wait()

        for b in range(NBUF):
            fetch_idx(b, b)
        fire_gathers(0, 0)

        def step(c, b, first):
            bp = (b - 1) % NBUF
            if not first:
                wait_write(b)
            fire_gathers(c, b)
            retire_chunk(c - 1, bp, last=False)

        for b in range(1, NBUF):
            step(b, b, True)

        def body(g, _):
            for b in range(NBUF):
                step(g * NBUF + b, b, False)
            return _

        lax.fori_loop(1, steps // NBUF, body, None)

        retire_chunk(steps - 1, (NBUF - 1) % NBUF, last=True)
        for b in range(NBUF):
            wait_write(b)

    return gather_kernel


_SLAB_N = NSENT // SLABS
_slab_kernel = _make_gather_kernel(_SLAB_N)


def kernel(x, table):
    idx = jnp.pad(x.astype(jnp.int32), ((0, 0), (0, SEQ_PAD - SEQ)))
    outs = [
        _slab_kernel(lax.slice_in_dim(idx, i * _SLAB_N, (i + 1) * _SLAB_N), table)
        for i in range(SLABS)
    ]
    return jnp.concatenate(outs, axis=0)


# speedup vs baseline: 1.5163x; 1.5163x over previous
import functools

import jax
import jax.numpy as jnp
from jax import lax
from jax.experimental import pallas as pl
from jax.experimental.pallas import tpu as pltpu
from jax.experimental.pallas import tpu_sc as plsc

VOCAB = 60000
EMBED_DIM = 128
SEQ = 50
NSENT = 16384
SEQ_PAD = 64
SEQ_OUT = 56
NBUF = 4

_info = plsc.get_sparse_core_info()
_NC, _NS = _info.num_cores, _info.num_subcores
_NW = _NC * _NS

_PER_W = NSENT // _NW
_CH = 4
_STEPS = _PER_W // _CH

_mesh = plsc.VectorSubcoreMesh(core_axis_name="c", subcore_axis_name="s")


@functools.partial(
    pl.kernel,
    mesh=_mesh,
    out_type=jax.ShapeDtypeStruct((NSENT, SEQ_OUT, EMBED_DIM), jnp.float32),
    scratch_types=[
        pltpu.VMEM((NBUF, _CH, SEQ_PAD), jnp.int32),
        pltpu.VMEM((NBUF, _CH, SEQ_OUT, EMBED_DIM), jnp.float32),
        [pltpu.SemaphoreType.DMA] * NBUF,
        [pltpu.SemaphoreType.DMA] * NBUF,
        [pltpu.SemaphoreType.DMA] * NBUF,
    ],
)
def _gather_kernel(idx_hbm, table_hbm, out_hbm, idx_v, rows_v, sg, so, si):
    wid = lax.axis_index("s") * _NC + lax.axis_index("c")
    base_sent = wid * _PER_W

    def fetch_idx(c, b):
        pltpu.async_copy(
            idx_hbm.at[pl.ds(base_sent + c * _CH, _CH)], idx_v.at[b], si[b]
        )

    def fire_gathers(c, b):
        pltpu.make_async_copy(
            idx_hbm.at[pl.ds(base_sent, _CH)], idx_v.at[b], si[b]
        ).wait()
        for j in range(_CH):
            pltpu.async_copy(
                table_hbm.at[idx_v.at[b, j, pl.ds(0, SEQ)]],
                rows_v.at[b, j, pl.ds(0, SEQ)],
                sg[b],
            )

    def retire_chunk(c, b, last):
        for _ in range(_CH):
            pltpu.make_async_copy(
                table_hbm.at[idx_v.at[b, 0, pl.ds(0, SEQ)]],
                rows_v.at[b, 0, pl.ds(0, SEQ)],
                sg[b],
            ).wait()
        if not last:
            @pl.when(c + NBUF < _STEPS)
            def _():
                fetch_idx(c + NBUF, b)

        pltpu.async_copy(
            rows_v.at[b], out_hbm.at[pl.ds(base_sent + c * _CH, _CH)], so[b]
        )

    def wait_write(b):
        pltpu.make_async_copy(
            rows_v.at[b], out_hbm.at[pl.ds(base_sent, _CH)], so[b]
        ).wait()

    for b in range(NBUF):
        fetch_idx(b, b)
    fire_gathers(0, 0)

    def step(c, b, first):
        bp = (b - 1) % NBUF
        if not first:
            wait_write(b)
        fire_gathers(c, b)
        retire_chunk(c - 1, bp, last=False)

    for b in range(1, NBUF):
        step(b, b, True)

    def body(g, _):
        for b in range(NBUF):
            step(g * NBUF + b, b, False)
        return _

    lax.fori_loop(1, _STEPS // NBUF, body, None)

    retire_chunk(_STEPS - 1, (NBUF - 1) % NBUF, last=True)
    for b in range(NBUF):
        wait_write(b)


def kernel(x, table):
    idx = jnp.pad(x.astype(jnp.int32), ((0, 0), (0, SEQ_PAD - SEQ)))
    out = _gather_kernel(idx, table)
    return lax.slice_in_dim(out, 0, SEQ, axis=1)


# speedup vs baseline: 1.7737x; 1.1698x over previous
import functools

import jax
import jax.numpy as jnp
from jax import lax
from jax.experimental import pallas as pl
from jax.experimental.pallas import tpu as pltpu
from jax.experimental.pallas import tpu_sc as plsc

VOCAB = 60000
EMBED_DIM = 128
SEQ = 50
NSENT = 16384
SEQ_PAD = 64
NBUF = 4

_info = plsc.get_sparse_core_info()
_NC, _NS = _info.num_cores, _info.num_subcores
_NW = _NC * _NS

_PER_W = NSENT // _NW
_CH = 4
_STEPS = _PER_W // _CH

_mesh = plsc.VectorSubcoreMesh(core_axis_name="c", subcore_axis_name="s")


@functools.partial(
    pl.kernel,
    mesh=_mesh,
    out_type=jax.ShapeDtypeStruct((NSENT, SEQ, EMBED_DIM), jnp.float32),
    scratch_types=[
        pltpu.VMEM((NBUF, _CH, SEQ_PAD), jnp.int32),
        pltpu.VMEM((NBUF, _CH, SEQ, EMBED_DIM), jnp.float32),
        [pltpu.SemaphoreType.DMA] * NBUF,
        [pltpu.SemaphoreType.DMA] * NBUF,
        [pltpu.SemaphoreType.DMA] * NBUF,
    ],
)
def _gather_kernel(idx_hbm, table_hbm, out_hbm, idx_v, rows_v, sg, so, si):
    wid = lax.axis_index("s") * _NC + lax.axis_index("c")
    base_sent = wid * _PER_W

    def fetch_idx(c, b):
        pltpu.async_copy(
            idx_hbm.at[pl.ds(base_sent + c * _CH, _CH)], idx_v.at[b], si[b]
        )

    def fire_gathers(c, b):
        pltpu.make_async_copy(
            idx_hbm.at[pl.ds(base_sent, _CH)], idx_v.at[b], si[b]
        ).wait()
        for j in range(_CH):
            pltpu.async_copy(
                table_hbm.at[idx_v.at[b, j, pl.ds(0, SEQ)]],
                rows_v.at[b, j],
                sg[b],
            )

    def retire_chunk(c, b, last):
        for _ in range(_CH):
            pltpu.make_async_copy(
                table_hbm.at[idx_v.at[b, 0, pl.ds(0, SEQ)]],
                rows_v.at[b, 0],
                sg[b],
            ).wait()
        if not last:
            @pl.when(c + NBUF < _STEPS)
            def _():
                fetch_idx(c + NBUF, b)

        pltpu.async_copy(
            rows_v.at[b], out_hbm.at[pl.ds(base_sent + c * _CH, _CH)], so[b]
        )

    def wait_write(b):
        pltpu.make_async_copy(
            rows_v.at[b], out_hbm.at[pl.ds(base_sent, _CH)], so[b]
        ).wait()

    for b in range(NBUF):
        fetch_idx(b, b)
    fire_gathers(0, 0)

    def step(c, b, first):
        bp = (b - 1) % NBUF
        if not first:
            wait_write(b)
        fire_gathers(c, b)
        retire_chunk(c - 1, bp, last=False)

    for b in range(1, NBUF):
        step(b, b, True)

    def body(g, _):
        for b in range(NBUF):
            step(g * NBUF + b, b, False)
        return _

    lax.fori_loop(1, _STEPS // NBUF, body, None)

    retire_chunk(_STEPS - 1, (NBUF - 1) % NBUF, last=True)
    for b in range(NBUF):
        wait_write(b)


def kernel(x, table):
    idx = jnp.pad(x.astype(jnp.int32), ((0, 0), (0, SEQ_PAD - SEQ)))
    return _gather_kernel(idx, table)
